# baseline (device time: 70517 ns/iter reference)
import jax
import jax.numpy as jnp
from jax import lax
from jax.experimental import pallas as pl
from jax.experimental.pallas import tpu as pltpu


def kernel(Q, K, V):
    B, S, H, D = Q.shape
    BH = B * H
    scale = D ** -0.5

    Qb = jnp.transpose(Q, (0, 2, 1, 3)).reshape(BH, S, D).astype(jnp.bfloat16)
    Kb = jnp.transpose(K, (0, 2, 1, 3)).reshape(BH, S, D).astype(jnp.bfloat16)
    Vb = jnp.transpose(V, (0, 2, 1, 3)).reshape(BH, S, D).astype(jnp.bfloat16)

    def body(q_ref, k_ref, v_ref, o_ref, kbuf, vbuf, send_sems, recv_sems):
        my_x = lax.axis_index("x")
        my_y = lax.axis_index("y")
        peer = (my_x, 1 - my_y)

        barrier_sem = pltpu.get_barrier_semaphore()
        pl.semaphore_signal(
            barrier_sem, inc=1, device_id=peer,
            device_id_type=pl.DeviceIdType.MESH,
        )
        pl.semaphore_wait(barrier_sem, 1)

        rk = pltpu.make_async_remote_copy(
            src_ref=k_ref, dst_ref=kbuf,
            send_sem=send_sems.at[0], recv_sem=recv_sems.at[0],
            device_id=peer, device_id_type=pl.DeviceIdType.MESH,
        )
        rk.start()
        rv = pltpu.make_async_remote_copy(
            src_ref=v_ref, dst_ref=vbuf,
            send_sem=send_sems.at[1], recv_sem=recv_sems.at[1],
            device_id=peer, device_id_type=pl.DeviceIdType.MESH,
        )
        rv.start()
        rk.wait()
        rv.wait()

        for i in range(BH):
            q = q_ref[i]
            s0 = lax.dot_general(
                q, k_ref[i], (((1,), (1,)), ((), ())),
                preferred_element_type=jnp.float32,
            )
            s1 = lax.dot_general(
                q, kbuf[i], (((1,), (1,)), ((), ())),
                preferred_element_type=jnp.float32,
            )
            s = jnp.concatenate([s0, s1], axis=1) * scale
            m = jnp.max(s, axis=1, keepdims=True)
            p = jnp.exp(s - m)
            l = jnp.sum(p, axis=1, keepdims=True)
            p = (p / l).astype(jnp.bfloat16)
            o0 = lax.dot_general(
                p[:, :S], v_ref[i], (((1,), (0,)), ((), ())),
                preferred_element_type=jnp.float32,
            )
            o1 = lax.dot_general(
                p[:, S:], vbuf[i], (((1,), (0,)), ((), ())),
                preferred_element_type=jnp.float32,
            )
            o_ref[i] = o0 + o1

    out = pl.pallas_call(
        body,
        out_shape=jax.ShapeDtypeStruct((BH, S, D), jnp.float32),
        in_specs=[pl.BlockSpec(memory_space=pltpu.VMEM)] * 3,
        out_specs=pl.BlockSpec(memory_space=pltpu.VMEM),
        scratch_shapes=[
            pltpu.VMEM((BH, S, D), jnp.bfloat16),
            pltpu.VMEM((BH, S, D), jnp.bfloat16),
            pltpu.SemaphoreType.DMA((2,)),
            pltpu.SemaphoreType.DMA((2,)),
        ],
        compiler_params=pltpu.CompilerParams(collective_id=0),
    )(Qb, Kb, Vb)

    return out.reshape(B, H, S, D).transpose(0, 2, 1, 3)


# device time: 67399 ns/iter; 1.0463x vs baseline; 1.0463x over previous
import jax
import jax.numpy as jnp
from jax import lax
from jax.experimental import pallas as pl
from jax.experimental.pallas import tpu as pltpu


def kernel(Q, K, V):
    B, S, H, D = Q.shape
    BH = B * H
    scale = D ** -0.5

    Qb = (jnp.transpose(Q, (0, 2, 1, 3)).reshape(BH, S, D) * scale).astype(
        jnp.bfloat16
    )
    Kb = jnp.transpose(K, (0, 2, 1, 3)).reshape(BH, S, D).astype(jnp.bfloat16)
    Vb = jnp.transpose(V, (0, 2, 1, 3)).reshape(BH, S, D).astype(jnp.bfloat16)

    def body(q_ref, k_ref, v_ref, o_ref, kbuf, vbuf, oacc, lacc,
             send_sems, recv_sems):
        my_x = lax.axis_index("x")
        my_y = lax.axis_index("y")
        peer = (my_x, 1 - my_y)

        barrier_sem = pltpu.get_barrier_semaphore()
        pl.semaphore_signal(
            barrier_sem, inc=1, device_id=peer,
            device_id_type=pl.DeviceIdType.MESH,
        )
        pl.semaphore_wait(barrier_sem, 1)

        rk = pltpu.make_async_remote_copy(
            src_ref=k_ref, dst_ref=kbuf,
            send_sem=send_sems.at[0], recv_sem=recv_sems.at[0],
            device_id=peer, device_id_type=pl.DeviceIdType.MESH,
        )
        rk.start()
        rv = pltpu.make_async_remote_copy(
            src_ref=v_ref, dst_ref=vbuf,
            send_sem=send_sems.at[1], recv_sem=recv_sems.at[1],
            device_id=peer, device_id_type=pl.DeviceIdType.MESH,
        )
        rv.start()

        ones = jnp.ones((S,), jnp.bfloat16)

        for i in range(BH):
            p0 = jnp.exp(
                lax.dot_general(
                    q_ref[i], k_ref[i], (((1,), (1,)), ((), ())),
                    preferred_element_type=jnp.float32,
                )
            ).astype(jnp.bfloat16)
            lacc[i] = lax.dot_general(
                p0, ones, (((1,), (0,)), ((), ())),
                preferred_element_type=jnp.float32,
            )
            oacc[i] = lax.dot_general(
                p0, v_ref[i], (((1,), (0,)), ((), ())),
                preferred_element_type=jnp.float32,
            )

        rk.wait()
        rv.wait()
        for i in range(BH):
            p1 = jnp.exp(
                lax.dot_general(
                    q_ref[i], kbuf[i], (((1,), (1,)), ((), ())),
                    preferred_element_type=jnp.float32,
                )
            ).astype(jnp.bfloat16)
            l1 = lax.dot_general(
                p1, ones, (((1,), (0,)), ((), ())),
                preferred_element_type=jnp.float32,
            )
            o1 = lax.dot_general(
                p1, vbuf[i], (((1,), (0,)), ((), ())),
                preferred_element_type=jnp.float32,
            )
            o_ref[i] = (oacc[i] + o1) / (lacc[i] + l1)[:, None]

    out = pl.pallas_call(
        body,
        out_shape=jax.ShapeDtypeStruct((BH, S, D), jnp.float32),
        in_specs=[pl.BlockSpec(memory_space=pltpu.VMEM)] * 3,
        out_specs=pl.BlockSpec(memory_space=pltpu.VMEM),
        scratch_shapes=[
            pltpu.VMEM((BH, S, D), jnp.bfloat16),
            pltpu.VMEM((BH, S, D), jnp.bfloat16),
            pltpu.VMEM((BH, S, D), jnp.float32),
            pltpu.VMEM((BH, S), jnp.float32),
            pltpu.SemaphoreType.DMA((2,)),
            pltpu.SemaphoreType.DMA((2,)),
        ],
        compiler_params=pltpu.CompilerParams(collective_id=0),
    )(Qb, Kb, Vb)

    return out.reshape(B, H, S, D).transpose(0, 2, 1, 3)


# device time: 61330 ns/iter; 1.1498x vs baseline; 1.0990x over previous
import jax
import jax.numpy as jnp
from jax import lax
from jax.experimental import pallas as pl
from jax.experimental.pallas import tpu as pltpu


def kernel(Q, K, V):
    B, S, H, D = Q.shape
    BH = B * H
    HALF = BH // 2
    scale = D ** -0.5

    x_idx = lax.axis_index("x")

    def prep(A, s):
        At = jnp.transpose(A, (0, 2, 1, 3)).reshape(BH, S, D)
        Ah = lax.dynamic_slice_in_dim(At, x_idx * HALF, HALF, axis=0)
        return (Ah * s).astype(jnp.bfloat16)

    Qb = prep(Q, scale)
    Kb = prep(K, 1.0)
    Vb = prep(V, 1.0)

    def body(q_ref, k_ref, v_ref, o_ref, kbuf, vbuf, oacc, lacc,
             kv_send, kv_recv, o_send, o_recv):
        my_x = lax.axis_index("x")
        my_y = lax.axis_index("y")
        ypeer = (my_x, 1 - my_y)
        xpeer = (1 - my_x, my_y)
        base = my_x * HALF

        barrier_sem = pltpu.get_barrier_semaphore()
        for peer in (ypeer, xpeer):
            pl.semaphore_signal(
                barrier_sem, inc=1, device_id=peer,
                device_id_type=pl.DeviceIdType.MESH,
            )
        pl.semaphore_wait(barrier_sem, 2)

        rk = pltpu.make_async_remote_copy(
            src_ref=k_ref, dst_ref=kbuf,
            send_sem=kv_send.at[0], recv_sem=kv_recv.at[0],
            device_id=ypeer, device_id_type=pl.DeviceIdType.MESH,
        )
        rk.start()
        rv = pltpu.make_async_remote_copy(
            src_ref=v_ref, dst_ref=vbuf,
            send_sem=kv_send.at[1], recv_sem=kv_recv.at[1],
            device_id=ypeer, device_id_type=pl.DeviceIdType.MESH,
        )
        rv.start()

        ones = jnp.ones((S,), jnp.bfloat16)

        def phase(kref, vref, j):
            p = jnp.exp(
                lax.dot_general(
                    q_ref[j], kref[j], (((1,), (1,)), ((), ())),
                    preferred_element_type=jnp.float32,
                )
            ).astype(jnp.bfloat16)
            l = lax.dot_general(
                p, ones, (((1,), (0,)), ((), ())),
                preferred_element_type=jnp.float32,
            )
            o = lax.dot_general(
                p, vref[j], (((1,), (0,)), ((), ())),
                preferred_element_type=jnp.float32,
            )
            return l, o

        for j in range(HALF):
            l0, o0 = phase(k_ref, v_ref, j)
            lacc[j] = l0
            oacc[j] = o0

        rk.wait()
        rv.wait()
        sends = []
        for j in range(HALF):
            l1, o1 = phase(kbuf, vbuf, j)
            o_ref[base + j] = (oacc[j] + o1) / (lacc[j] + l1)[:, None]
            ro = pltpu.make_async_remote_copy(
                src_ref=o_ref.at[base + j], dst_ref=o_ref.at[base + j],
                send_sem=o_send.at[j], recv_sem=o_recv.at[j],
                device_id=xpeer, device_id_type=pl.DeviceIdType.MESH,
            )
            ro.start()
            sends.append(ro)
        for ro in sends:
            ro.wait()

    out = pl.pallas_call(
        body,
        out_shape=jax.ShapeDtypeStruct((BH, S, D), jnp.float32),
        in_specs=[pl.BlockSpec(memory_space=pltpu.VMEM)] * 3,
        out_specs=pl.BlockSpec(memory_space=pltpu.VMEM),
        scratch_shapes=[
            pltpu.VMEM((HALF, S, D), jnp.bfloat16),
            pltpu.VMEM((HALF, S, D), jnp.bfloat16),
            pltpu.VMEM((HALF, S, D), jnp.float32),
            pltpu.VMEM((HALF, S), jnp.float32),
            pltpu.SemaphoreType.DMA((2,)),
            pltpu.SemaphoreType.DMA((2,)),
            pltpu.SemaphoreType.DMA((HALF,)),
            pltpu.SemaphoreType.DMA((HALF,)),
        ],
        compiler_params=pltpu.CompilerParams(collective_id=0),
    )(Qb, Kb, Vb)

    return out.reshape(B, H, S, D).transpose(0, 2, 1, 3)


# device time: 51772 ns/iter; 1.3621x vs baseline; 1.1846x over previous
import jax
import jax.numpy as jnp
from jax import lax
from jax.experimental import pallas as pl
from jax.experimental.pallas import tpu as pltpu


def kernel(Q, K, V):
    B, S, H, D = Q.shape
    BH = B * H
    scale = D ** -0.5

    x_idx = lax.axis_index("x")

    def prep(A, s):
        Ah = lax.dynamic_slice_in_dim(A, x_idx, 1, axis=0)[0]
        return jnp.transpose((Ah * s).astype(jnp.bfloat16), (1, 0, 2))

    Qb = prep(Q, scale)
    KVb = jnp.concatenate([prep(K, 1.0), prep(V, 1.0)], axis=0)

    def body(q_ref, kv_ref, o_ref, kvbuf, oacc, lacc,
             kv_sems, o_sems):
        my_x = lax.axis_index("x")
        my_y = lax.axis_index("y")
        ypeer = (my_x, 1 - my_y)
        xpeer = (1 - my_x, my_y)
        base = my_x * H

        with jax.named_scope("barrier"):
            barrier_sem = pltpu.get_barrier_semaphore()
            for peer in (ypeer, xpeer):
                pl.semaphore_signal(
                    barrier_sem, inc=1, device_id=peer,
                    device_id_type=pl.DeviceIdType.MESH,
                )
            pl.semaphore_wait(barrier_sem, 2)

        rkv = pltpu.make_async_remote_copy(
            src_ref=kv_ref, dst_ref=kvbuf,
            send_sem=kv_sems.at[0], recv_sem=kv_sems.at[1],
            device_id=ypeer, device_id_type=pl.DeviceIdType.MESH,
        )
        rkv.start()

        ones = jnp.ones((S,), jnp.bfloat16)

        def phase(kvref, j):
            p = jnp.exp(
                lax.dot_general(
                    q_ref[j], kvref[j], (((1,), (1,)), ((), ())),
                    preferred_element_type=jnp.float32,
                )
            ).astype(jnp.bfloat16)
            l = lax.dot_general(
                p, ones, (((1,), (0,)), ((), ())),
                preferred_element_type=jnp.float32,
            )
            o = lax.dot_general(
                p, kvref[H + j], (((1,), (0,)), ((), ())),
                preferred_element_type=jnp.float32,
            )
            return l, o

        with jax.named_scope("local_phase"):
            for j in range(H):
                l0, o0 = phase(kv_ref, j)
                lacc[j] = l0
                oacc[j] = o0

        with jax.named_scope("kv_wait"):
            rkv.wait()

        with jax.named_scope("remote_phase"):
            for j in range(H):
                l1, o1 = phase(kvbuf, j)
                r = (oacc[j] + o1) / (lacc[j] + l1)[:, None]
                o_ref[base + j] = r.astype(jnp.bfloat16)

        with jax.named_scope("out_flush"):
            ro = pltpu.make_async_remote_copy(
                src_ref=o_ref.at[pl.ds(base, H)],
                dst_ref=o_ref.at[pl.ds(base, H)],
                send_sem=o_sems.at[0], recv_sem=o_sems.at[1],
                device_id=xpeer, device_id_type=pl.DeviceIdType.MESH,
            )
            ro.start()
            ro.wait()

    out = pl.pallas_call(
        body,
        out_shape=jax.ShapeDtypeStruct((BH, S, D), jnp.bfloat16),
        in_specs=[pl.BlockSpec(memory_space=pltpu.VMEM)] * 2,
        out_specs=pl.BlockSpec(memory_space=pltpu.VMEM),
        scratch_shapes=[
            pltpu.VMEM((2 * H, S, D), jnp.bfloat16),
            pltpu.VMEM((H, S, D), jnp.float32),
            pltpu.VMEM((H, S), jnp.float32),
            pltpu.SemaphoreType.DMA((2,)),
            pltpu.SemaphoreType.DMA((2,)),
        ],
        compiler_params=pltpu.CompilerParams(collective_id=0),
    )(Qb, KVb)

    return (
        out.reshape(B, H, S, D).transpose(0, 2, 1, 3).astype(jnp.float32)
    )


# device time: 32468 ns/iter; 2.1719x vs baseline; 1.5946x over previous
import jax
import jax.numpy as jnp
from jax import lax
from jax.experimental import pallas as pl
from jax.experimental.pallas import tpu as pltpu


def kernel(Q, K, V):
    B, S, H, D = Q.shape
    BH = B * H
    scale = D ** -0.5

    x_idx = lax.axis_index("x")

    def prep(A, s):
        Ah = lax.dynamic_slice_in_dim(A, x_idx, 1, axis=0)[0]
        return jnp.transpose((Ah * s).astype(jnp.bfloat16), (1, 2, 0))

    Qb = prep(Q, scale)
    KVb = jnp.concatenate([prep(K, 1.0), prep(V, 1.0)], axis=0)

    def body(q_ref, kv_ref, o_ref, kvbuf, oacc, lacc, kv_sems, o_sems):
        my_x = lax.axis_index("x")
        my_y = lax.axis_index("y")
        ypeer = (my_x, 1 - my_y)
        xpeer = (1 - my_x, my_y)
        base = my_x * H

        with jax.named_scope("barrier"):
            barrier_sem = pltpu.get_barrier_semaphore()
            for peer in (ypeer, xpeer):
                pl.semaphore_signal(
                    barrier_sem, inc=1, device_id=peer,
                    device_id_type=pl.DeviceIdType.MESH,
                )
            pl.semaphore_wait(barrier_sem, 2)

        rkv = pltpu.make_async_remote_copy(
            src_ref=kv_ref, dst_ref=kvbuf,
            send_sem=kv_sems.at[0], recv_sem=kv_sems.at[1],
            device_id=ypeer, device_id_type=pl.DeviceIdType.MESH,
        )
        rkv.start()

        ones = jnp.ones((S,), jnp.bfloat16)

        def phase(kvref, j):
            pT = jnp.exp(
                lax.dot_general(
                    kvref[j], q_ref[j], (((0,), (0,)), ((), ())),
                    preferred_element_type=jnp.float32,
                )
            ).astype(jnp.bfloat16)
            l = lax.dot_general(
                pT, ones, (((0,), (0,)), ((), ())),
                preferred_element_type=jnp.float32,
            )
            oT = lax.dot_general(
                kvref[H + j], pT, (((1,), (0,)), ((), ())),
                preferred_element_type=jnp.float32,
            )
            return l, oT

        with jax.named_scope("local_phase"):
            for j in range(H):
                l0, o0 = phase(kv_ref, j)
                lacc[j] = l0
                oacc[j] = o0

        with jax.named_scope("kv_wait"):
            rkv.wait()

        with jax.named_scope("remote_phase"):
            for j in range(H):
                l1, o1 = phase(kvbuf, j)
                r = (oacc[j] + o1) / (lacc[j] + l1)[None, :]
                o_ref[base + j] = r.astype(jnp.bfloat16)

        with jax.named_scope("out_flush"):
            ro = pltpu.make_async_remote_copy(
                src_ref=o_ref.at[pl.ds(base, H)],
                dst_ref=o_ref.at[pl.ds(base, H)],
                send_sem=o_sems.at[0], recv_sem=o_sems.at[1],
                device_id=xpeer, device_id_type=pl.DeviceIdType.MESH,
            )
            ro.start()
            ro.wait()

    out = pl.pallas_call(
        body,
        out_shape=jax.ShapeDtypeStruct((BH, D, S), jnp.bfloat16),
        in_specs=[pl.BlockSpec(memory_space=pltpu.VMEM)] * 2,
        out_specs=pl.BlockSpec(memory_space=pltpu.VMEM),
        scratch_shapes=[
            pltpu.VMEM((2 * H, D, S), jnp.bfloat16),
            pltpu.VMEM((H, D, S), jnp.float32),
            pltpu.VMEM((H, S), jnp.float32),
            pltpu.SemaphoreType.DMA((2,)),
            pltpu.SemaphoreType.DMA((2,)),
        ],
        compiler_params=pltpu.CompilerParams(collective_id=0),
    )(Qb, KVb)

    return (
        out.reshape(B, H, D, S).transpose(0, 3, 1, 2).astype(jnp.float32)
    )


# device time: 24013 ns/iter; 2.9366x vs baseline; 1.3521x over previous
import jax
import jax.numpy as jnp
from jax import lax
from jax.experimental import pallas as pl
from jax.experimental.pallas import tpu as pltpu


def kernel(Q, K, V):
    B, S, H, D = Q.shape
    BH = B * H
    scale = D ** -0.5

    x_idx = lax.axis_index("x")

    def prep(A, s):
        Ah = lax.dynamic_slice_in_dim(A, x_idx, 1, axis=0)[0]
        return jnp.transpose((Ah * s).astype(jnp.bfloat16), (1, 2, 0))

    Qb = prep(Q, scale)
    KVb = jnp.stack([prep(K, 1.0), prep(V, 1.0)], axis=1).reshape(
        2 * H, D, S
    )

    def body(q_ref, kv_ref, o_ref, kvbuf, oacc, lacc,
             kv_send, kv_recv, o_send, o_recv):
        my_x = lax.axis_index("x")
        my_y = lax.axis_index("y")
        ypeer = (my_x, 1 - my_y)
        xpeer = (1 - my_x, my_y)
        base = my_x * H

        with jax.named_scope("barrier"):
            barrier_sem = pltpu.get_barrier_semaphore()
            for peer in (ypeer, xpeer):
                pl.semaphore_signal(
                    barrier_sem, inc=1, device_id=peer,
                    device_id_type=pl.DeviceIdType.MESH,
                )
            pl.semaphore_wait(barrier_sem, 2)

        rkvs = []
        for j in range(H):
            rkv = pltpu.make_async_remote_copy(
                src_ref=kv_ref.at[pl.ds(2 * j, 2)],
                dst_ref=kvbuf.at[pl.ds(2 * j, 2)],
                send_sem=kv_send.at[j], recv_sem=kv_recv.at[j],
                device_id=ypeer, device_id_type=pl.DeviceIdType.MESH,
            )
            rkv.start()
            rkvs.append(rkv)

        ones = jnp.ones((S,), jnp.bfloat16)

        def phase(kvref, j):
            pT = jnp.exp(
                lax.dot_general(
                    kvref[2 * j], q_ref[j], (((0,), (0,)), ((), ())),
                    preferred_element_type=jnp.float32,
                )
            ).astype(jnp.bfloat16)
            l = lax.dot_general(
                pT, ones, (((0,), (0,)), ((), ())),
                preferred_element_type=jnp.float32,
            )
            oT = lax.dot_general(
                kvref[2 * j + 1], pT, (((1,), (0,)), ((), ())),
                preferred_element_type=jnp.float32,
            )
            return l, oT

        with jax.named_scope("local_phase"):
            for j in range(H):
                l0, o0 = phase(kv_ref, j)
                lacc[j] = l0
                oacc[j] = o0

        ros = []
        with jax.named_scope("remote_phase"):
            for j in range(H):
                rkvs[j].wait_recv()
                l1, o1 = phase(kvbuf, j)
                r = (oacc[j] + o1) / (lacc[j] + l1)[None, :]
                o_ref[base + j] = r.astype(jnp.bfloat16)
                ro = pltpu.make_async_remote_copy(
                    src_ref=o_ref.at[base + j],
                    dst_ref=o_ref.at[base + j],
                    send_sem=o_send.at[j], recv_sem=o_recv.at[j],
                    device_id=xpeer, device_id_type=pl.DeviceIdType.MESH,
                )
                ro.start()
                ros.append(ro)

        with jax.named_scope("out_flush"):
            for j in range(H):
                ros[j].wait_recv()
            for j in range(H):
                ros[j].wait_send()
                rkvs[j].wait_send()

    out = pl.pallas_call(
        body,
        out_shape=jax.ShapeDtypeStruct((BH, D, S), jnp.bfloat16),
        in_specs=[pl.BlockSpec(memory_space=pltpu.VMEM)] * 2,
        out_specs=pl.BlockSpec(memory_space=pltpu.VMEM),
        scratch_shapes=[
            pltpu.VMEM((2 * H, D, S), jnp.bfloat16),
            pltpu.VMEM((H, D, S), jnp.float32),
            pltpu.VMEM((H, S), jnp.float32),
            pltpu.SemaphoreType.DMA((H,)),
            pltpu.SemaphoreType.DMA((H,)),
            pltpu.SemaphoreType.DMA((H,)),
            pltpu.SemaphoreType.DMA((H,)),
        ],
        compiler_params=pltpu.CompilerParams(collective_id=0),
    )(Qb, KVb)

    return (
        out.reshape(B, H, D, S).transpose(0, 3, 1, 2).astype(jnp.float32)
    )


# device time: 23415 ns/iter; 3.0116x vs baseline; 1.0255x over previous
import jax
import jax.numpy as jnp
from jax import lax
from jax.experimental import pallas as pl
from jax.experimental.pallas import tpu as pltpu


def kernel(Q, K, V):
    B, S, H, D = Q.shape
    BH = B * H
    scale = D ** -0.5

    x_idx = lax.axis_index("x")

    def prep(A, s):
        Ah = lax.dynamic_slice_in_dim(A, x_idx, 1, axis=0)[0]
        return jnp.transpose((Ah * s).astype(jnp.bfloat16), (1, 2, 0))

    Qb = prep(Q, scale)
    KVb = jnp.stack([prep(K, 1.0), prep(V, 1.0)], axis=1).reshape(
        2 * H, D, S
    )

    def body(q_ref, kv_ref, o_ref, kvbuf, oacc, lacc,
             kv_send, kv_recv, o_send, o_recv, xbar_sem):
        my_x = lax.axis_index("x")
        my_y = lax.axis_index("y")
        ypeer = (my_x, 1 - my_y)
        xpeer = (1 - my_x, my_y)
        base = my_x * H

        with jax.named_scope("barrier"):
            barrier_sem = pltpu.get_barrier_semaphore()
            pl.semaphore_signal(
                barrier_sem, inc=1, device_id=ypeer,
                device_id_type=pl.DeviceIdType.MESH,
            )
            pl.semaphore_wait(barrier_sem, 1)

        rkvs = []
        for j in range(H):
            rkv = pltpu.make_async_remote_copy(
                src_ref=kv_ref.at[pl.ds(2 * j, 2)],
                dst_ref=kvbuf.at[pl.ds(2 * j, 2)],
                send_sem=kv_send.at[j], recv_sem=kv_recv.at[j],
                device_id=ypeer, device_id_type=pl.DeviceIdType.MESH,
            )
            rkv.start()
            rkvs.append(rkv)

        pl.semaphore_signal(
            xbar_sem, inc=1, device_id=xpeer,
            device_id_type=pl.DeviceIdType.MESH,
        )

        ones = jnp.ones((S,), jnp.bfloat16)

        def phase(kvref, j):
            pT = jnp.exp(
                lax.dot_general(
                    kvref[2 * j], q_ref[j], (((0,), (0,)), ((), ())),
                    preferred_element_type=jnp.float32,
                )
            ).astype(jnp.bfloat16)
            l = lax.dot_general(
                pT, ones, (((0,), (0,)), ((), ())),
                preferred_element_type=jnp.float32,
            )
            oT = lax.dot_general(
                kvref[2 * j + 1], pT, (((1,), (0,)), ((), ())),
                preferred_element_type=jnp.float32,
            )
            return l, oT

        with jax.named_scope("local_phase"):
            for j in range(H):
                l0, o0 = phase(kv_ref, j)
                lacc[j] = l0
                oacc[j] = o0

        ros = []
        with jax.named_scope("remote_phase"):
            for j in range(H):
                rkvs[j].wait_recv()
                if j == 0:
                    pl.semaphore_wait(xbar_sem, 1)
                l1, o1 = phase(kvbuf, j)
                r = (oacc[j] + o1) / (lacc[j] + l1)[None, :]
                o_ref[base + j] = r.astype(jnp.bfloat16)
                ro = pltpu.make_async_remote_copy(
                    src_ref=o_ref.at[base + j],
                    dst_ref=o_ref.at[base + j],
                    send_sem=o_send.at[j], recv_sem=o_recv.at[j],
                    device_id=xpeer, device_id_type=pl.DeviceIdType.MESH,
                )
                ro.start()
                ros.append(ro)

        with jax.named_scope("out_flush"):
            for j in range(H):
                ros[j].wait_recv()
            for j in range(H):
                ros[j].wait_send()
                rkvs[j].wait_send()

    out = pl.pallas_call(
        body,
        out_shape=jax.ShapeDtypeStruct((BH, D, S), jnp.bfloat16),
        in_specs=[pl.BlockSpec(memory_space=pltpu.VMEM)] * 2,
        out_specs=pl.BlockSpec(memory_space=pltpu.VMEM),
        scratch_shapes=[
            pltpu.VMEM((2 * H, D, S), jnp.bfloat16),
            pltpu.VMEM((H, D, S), jnp.float32),
            pltpu.VMEM((H, S), jnp.float32),
            pltpu.SemaphoreType.DMA((H,)),
            pltpu.SemaphoreType.DMA((H,)),
            pltpu.SemaphoreType.DMA((H,)),
            pltpu.SemaphoreType.DMA((H,)),
            pltpu.SemaphoreType.REGULAR,
        ],
        compiler_params=pltpu.CompilerParams(collective_id=0),
    )(Qb, KVb)

    return out.reshape(B, H, D, S).transpose(0, 3, 1, 2)


# device time: 23347 ns/iter; 3.0204x vs baseline; 1.0029x over previous
import jax
import jax.numpy as jnp
from jax import lax
from jax.experimental import pallas as pl
from jax.experimental.pallas import tpu as pltpu


def kernel(Q, K, V):
    B, S, H, D = Q.shape
    BH = B * H
    scale = D ** -0.5

    x_idx = lax.axis_index("x")

    def prep(A, s):
        Ah = lax.dynamic_slice_in_dim(A, x_idx, 1, axis=0)[0]
        return jnp.transpose((Ah * s).astype(jnp.bfloat16), (1, 2, 0))

    Qb = prep(Q, scale)
    Kb = prep(K, 1.0)
    Vb = prep(V, 1.0)

    def body(q_ref, k_ref, v_ref, o_ref, kbuf, vbuf, oacc, lacc,
             kv_send, kv_recv, o_send, o_recv, xbar_sem):
        my_x = lax.axis_index("x")
        my_y = lax.axis_index("y")
        ypeer = (my_x, 1 - my_y)
        xpeer = (1 - my_x, my_y)
        base = my_x * H

        with jax.named_scope("barrier"):
            barrier_sem = pltpu.get_barrier_semaphore()
            pl.semaphore_signal(
                barrier_sem, inc=1, device_id=ypeer,
                device_id_type=pl.DeviceIdType.MESH,
            )
            pl.semaphore_wait(barrier_sem, 1)

        rkvs = []
        for j in range(H):
            rk = pltpu.make_async_remote_copy(
                src_ref=k_ref.at[j], dst_ref=kbuf.at[j],
                send_sem=kv_send.at[2 * j], recv_sem=kv_recv.at[2 * j],
                device_id=ypeer, device_id_type=pl.DeviceIdType.MESH,
            )
            rk.start()
            rv = pltpu.make_async_remote_copy(
                src_ref=v_ref.at[j], dst_ref=vbuf.at[j],
                send_sem=kv_send.at[2 * j + 1],
                recv_sem=kv_recv.at[2 * j + 1],
                device_id=ypeer, device_id_type=pl.DeviceIdType.MESH,
            )
            rv.start()
            rkvs.append((rk, rv))

        pl.semaphore_signal(
            xbar_sem, inc=1, device_id=xpeer,
            device_id_type=pl.DeviceIdType.MESH,
        )

        ones = jnp.ones((S,), jnp.bfloat16)

        def phase(kref, vref, j):
            pT = jnp.exp(
                lax.dot_general(
                    kref[j], q_ref[j], (((0,), (0,)), ((), ())),
                    preferred_element_type=jnp.float32,
                )
            ).astype(jnp.bfloat16)
            l = lax.dot_general(
                pT, ones, (((0,), (0,)), ((), ())),
                preferred_element_type=jnp.float32,
            )
            oT = lax.dot_general(
                vref[j], pT, (((1,), (0,)), ((), ())),
                preferred_element_type=jnp.float32,
            )
            return l, oT

        with jax.named_scope("local_phase"):
            for j in range(H):
                l0, o0 = phase(k_ref, v_ref, j)
                lacc[j] = l0
                oacc[j] = o0

        ros = []
        with jax.named_scope("remote_phase"):
            for j in range(H):
                rkvs[j][0].wait_recv()
                rkvs[j][1].wait_recv()
                if j == 0:
                    pl.semaphore_wait(xbar_sem, 1)
                l1, o1 = phase(kbuf, vbuf, j)
                r = (oacc[j] + o1) / (lacc[j] + l1)[None, :]
                o_ref[base + j] = r.astype(jnp.bfloat16)
                ro = pltpu.make_async_remote_copy(
                    src_ref=o_ref.at[base + j],
                    dst_ref=o_ref.at[base + j],
                    send_sem=o_send.at[j], recv_sem=o_recv.at[j],
                    device_id=xpeer, device_id_type=pl.DeviceIdType.MESH,
                )
                ro.start()
                ros.append(ro)

        with jax.named_scope("out_flush"):
            for j in range(H):
                ros[j].wait_recv()
            for j in range(H):
                ros[j].wait_send()
                rkvs[j][0].wait_send()
                rkvs[j][1].wait_send()

    out = pl.pallas_call(
        body,
        out_shape=jax.ShapeDtypeStruct((BH, D, S), jnp.bfloat16),
        in_specs=[pl.BlockSpec(memory_space=pltpu.VMEM)] * 3,
        out_specs=pl.BlockSpec(memory_space=pltpu.VMEM),
        scratch_shapes=[
            pltpu.VMEM((H, D, S), jnp.bfloat16),
            pltpu.VMEM((H, D, S), jnp.bfloat16),
            pltpu.VMEM((H, D, S), jnp.float32),
            pltpu.VMEM((H, S), jnp.float32),
            pltpu.SemaphoreType.DMA((2 * H,)),
            pltpu.SemaphoreType.DMA((2 * H,)),
            pltpu.SemaphoreType.DMA((H,)),
            pltpu.SemaphoreType.DMA((H,)),
            pltpu.SemaphoreType.REGULAR,
        ],
        compiler_params=pltpu.CompilerParams(collective_id=0),
    )(Qb, Kb, Vb)

    return out.reshape(B, H, D, S).transpose(0, 3, 1, 2)


# device time: 22305 ns/iter; 3.1615x vs baseline; 1.0467x over previous
import jax
import jax.numpy as jnp
from jax import lax
from jax.experimental import pallas as pl
from jax.experimental.pallas import tpu as pltpu


def kernel(Q, K, V):
    B, S, H, D = Q.shape
    BH = B * H
    scale = D ** -0.5

    x_idx = lax.axis_index("x")

    def prep(A, s):
        Ah = lax.dynamic_slice_in_dim(A, x_idx, 1, axis=0)[0]
        return jnp.transpose((Ah * s).astype(jnp.bfloat16), (1, 2, 0))

    Qb = prep(Q, scale)
    Kb = prep(K, 1.0)
    Vb = prep(V, 1.0)

    def body(q_ref, k_ref, v_ref, o_ref, oacc, lacc, opart, obuf, lbuf,
             op_send, op_recv, lp_send, lp_recv, o_send, o_recv,
             xbar_sem):
        my_x = lax.axis_index("x")
        my_y = lax.axis_index("y")
        ypeer = (my_x, 1 - my_y)
        xpeer = (1 - my_x, my_y)
        base = my_x * H

        with jax.named_scope("barrier"):
            barrier_sem = pltpu.get_barrier_semaphore()
            pl.semaphore_signal(
                barrier_sem, inc=1, device_id=ypeer,
                device_id_type=pl.DeviceIdType.MESH,
            )
            pl.semaphore_wait(barrier_sem, 1)

        pl.semaphore_signal(
            xbar_sem, inc=1, device_id=xpeer,
            device_id_type=pl.DeviceIdType.MESH,
        )

        ones = jnp.ones((S,), jnp.bfloat16)

        sends = []
        with jax.named_scope("partial_phase"):
            for j in range(H):
                pT = jnp.exp(
                    lax.dot_general(
                        k_ref[j], q_ref[j], (((0,), (0,)), ((), ())),
                        preferred_element_type=jnp.float32,
                    )
                ).astype(jnp.bfloat16)
                lacc[j] = lax.dot_general(
                    pT, ones, (((0,), (0,)), ((), ())),
                    preferred_element_type=jnp.float32,
                )
                oT = lax.dot_general(
                    v_ref[j], pT, (((1,), (0,)), ((), ())),
                    preferred_element_type=jnp.float32,
                )
                oacc[j] = oT
                opart[j] = oT.astype(jnp.bfloat16)
                ro = pltpu.make_async_remote_copy(
                    src_ref=opart.at[j], dst_ref=obuf.at[j],
                    send_sem=op_send.at[j], recv_sem=op_recv.at[j],
                    device_id=ypeer, device_id_type=pl.DeviceIdType.MESH,
                )
                ro.start()
                rl = pltpu.make_async_remote_copy(
                    src_ref=lacc.at[j], dst_ref=lbuf.at[j],
                    send_sem=lp_send.at[j], recv_sem=lp_recv.at[j],
                    device_id=ypeer, device_id_type=pl.DeviceIdType.MESH,
                )
                rl.start()
                sends.append((ro, rl))

        ros = []
        with jax.named_scope("combine_phase"):
            for j in range(H):
                sends[j][0].wait_recv()
                sends[j][1].wait_recv()
                if j == 0:
                    pl.semaphore_wait(xbar_sem, 1)
                r = (oacc[j] + obuf[j]) / (lacc[j] + lbuf[j])[None, :]
                o_ref[base + j] = r.astype(jnp.bfloat16)
                ro = pltpu.make_async_remote_copy(
                    src_ref=o_ref.at[base + j],
                    dst_ref=o_ref.at[base + j],
                    send_sem=o_send.at[j], recv_sem=o_recv.at[j],
                    device_id=xpeer, device_id_type=pl.DeviceIdType.MESH,
                )
                ro.start()
                ros.append(ro)

        with jax.named_scope("out_flush"):
            for j in range(H):
                ros[j].wait_recv()
            for j in range(H):
                ros[j].wait_send()
                sends[j][0].wait_send()
                sends[j][1].wait_send()

    out = pl.pallas_call(
        body,
        out_shape=jax.ShapeDtypeStruct((BH, D, S), jnp.bfloat16),
        in_specs=[pl.BlockSpec(memory_space=pltpu.VMEM)] * 3,
        out_specs=pl.BlockSpec(memory_space=pltpu.VMEM),
        scratch_shapes=[
            pltpu.VMEM((H, D, S), jnp.float32),
            pltpu.VMEM((H, S), jnp.float32),
            pltpu.VMEM((H, D, S), jnp.bfloat16),
            pltpu.VMEM((H, D, S), jnp.bfloat16),
            pltpu.VMEM((H, S), jnp.float32),
            pltpu.SemaphoreType.DMA((H,)),
            pltpu.SemaphoreType.DMA((H,)),
            pltpu.SemaphoreType.DMA((H,)),
            pltpu.SemaphoreType.DMA((H,)),
            pltpu.SemaphoreType.DMA((H,)),
            pltpu.SemaphoreType.DMA((H,)),
            pltpu.SemaphoreType.REGULAR,
        ],
        compiler_params=pltpu.CompilerParams(collective_id=0),
    )(Qb, Kb, Vb)

    return out.reshape(B, H, D, S).transpose(0, 3, 1, 2)
